# baseline (speedup 1.0000x reference)
import functools

import jax
import jax.numpy as jnp
from jax import lax
from jax.experimental import pallas as pl
from jax.experimental.pallas import tpu as pltpu
from jax.experimental.pallas import tpu_sc as plsc

BATCH = 16384
HIDDEN = 32

_NC = 2
_NS = 16
_NW = _NC * _NS
_BPW = BATCH // _NW
_CHUNK = 128
_NCHUNK = _BPW // _CHUNK
_ROWS_PER_W = _BPW // _CHUNK


def _gather_body(uidx, iidx, jidx, uemb, iemb, ibias,
                 u_out, i_out, j_out, ib_out, jb_out,
                 idxu_v, idxi_v, idxj_v, u_v, i_v, j_v, ib_v, jb_v, sem):
    wid = lax.axis_index("s") * _NC + lax.axis_index("c")
    row0 = wid * _ROWS_PER_W
    base = wid * _BPW

    pltpu.sync_copy(uidx.at[pl.ds(row0, _ROWS_PER_W)], idxu_v)
    pltpu.sync_copy(iidx.at[pl.ds(row0, _ROWS_PER_W)], idxi_v)
    pltpu.sync_copy(jidx.at[pl.ds(row0, _ROWS_PER_W)], idxj_v)

    copies = []
    for k in range(_NCHUNK):
        dst = pl.ds(k * _CHUNK, _CHUNK)
        copies.append(pltpu.async_copy(uemb.at[idxu_v.at[k]], u_v.at[dst], sem))
        copies.append(pltpu.async_copy(iemb.at[idxi_v.at[k]], i_v.at[dst], sem))
        copies.append(pltpu.async_copy(iemb.at[idxj_v.at[k]], j_v.at[dst], sem))
        copies.append(pltpu.async_copy(ibias.at[idxi_v.at[k]], ib_v.at[k], sem))
        copies.append(pltpu.async_copy(ibias.at[idxj_v.at[k]], jb_v.at[k], sem))
    for c in copies:
        c.wait()

    pltpu.sync_copy(u_v, u_out.at[pl.ds(base, _BPW)])
    pltpu.sync_copy(i_v, i_out.at[pl.ds(base, _BPW)])
    pltpu.sync_copy(j_v, j_out.at[pl.ds(base, _BPW)])
    pltpu.sync_copy(ib_v, ib_out.at[pl.ds(row0, _ROWS_PER_W)])
    pltpu.sync_copy(jb_v, jb_out.at[pl.ds(row0, _ROWS_PER_W)])


@functools.partial(jax.jit, static_argnames=())
def _sc_gather(uidx2, iidx2, jidx2, user_embedding, item_embedding, item_bias):
    mesh = plsc.VectorSubcoreMesh(core_axis_name="c", subcore_axis_name="s")
    f = functools.partial(
        pl.kernel,
        mesh=mesh,
        compiler_params=pltpu.CompilerParams(use_tc_tiling_on_sc=False),
        out_type=(
            jax.ShapeDtypeStruct((BATCH, HIDDEN), jnp.float32),
            jax.ShapeDtypeStruct((BATCH, HIDDEN), jnp.float32),
            jax.ShapeDtypeStruct((BATCH, HIDDEN), jnp.float32),
            jax.ShapeDtypeStruct((128, 128), jnp.float32),
            jax.ShapeDtypeStruct((128, 128), jnp.float32),
        ),
        scratch_types=[
            pltpu.VMEM((_ROWS_PER_W, 128), jnp.int32),
            pltpu.VMEM((_ROWS_PER_W, 128), jnp.int32),
            pltpu.VMEM((_ROWS_PER_W, 128), jnp.int32),
            pltpu.VMEM((_BPW, HIDDEN), jnp.float32),
            pltpu.VMEM((_BPW, HIDDEN), jnp.float32),
            pltpu.VMEM((_BPW, HIDDEN), jnp.float32),
            pltpu.VMEM((_ROWS_PER_W, 128), jnp.float32),
            pltpu.VMEM((_ROWS_PER_W, 128), jnp.float32),
            pltpu.SemaphoreType.DMA,
        ],
    )(_gather_body)
    return f(uidx2, iidx2, jidx2, user_embedding, item_embedding, item_bias)


def _loss_body(u_ref, i_ref, j_ref, ib_ref, jb_ref, out_ref):
    u = u_ref[...]
    i = i_ref[...]
    j = j_ref[...]
    dot = jnp.sum(u * (i - j), axis=1, keepdims=True)
    x = ib_ref[...] - jb_ref[...] + dot
    ls = jnp.minimum(x, 0.0) - jnp.log(1.0 + jnp.exp(-jnp.abs(x)))
    l2 = jnp.sum(u * u) + jnp.sum(i * i) + jnp.sum(j * j)
    out_ref[0, 0] = 0.0001 * l2 - jnp.mean(ls)


def _tc_loss(u_rows, i_rows, j_rows, ib, jb):
    return pl.pallas_call(
        _loss_body,
        out_shape=jax.ShapeDtypeStruct((1, 1), jnp.float32),
        out_specs=pl.BlockSpec(memory_space=pltpu.SMEM),
    )(u_rows, i_rows, j_rows, ib, jb)


def kernel(input, user_embedding, item_embedding, item_bias):
    idx = input.astype(jnp.int32)
    uidx2 = idx[:, 0].reshape(128, 128)
    iidx2 = idx[:, 1].reshape(128, 128)
    jidx2 = idx[:, 2].reshape(128, 128)
    u_rows, i_rows, j_rows, ib2, jb2 = _sc_gather(
        uidx2, iidx2, jidx2, user_embedding, item_embedding, item_bias)
    loss = _tc_loss(u_rows, i_rows, j_rows,
                    ib2.reshape(BATCH, 1), jb2.reshape(BATCH, 1))
    return loss.reshape(())


# speedup vs baseline: 1.0142x; 1.0142x over previous
import functools

import jax
import jax.numpy as jnp
from jax import lax
from jax.experimental import pallas as pl
from jax.experimental.pallas import tpu as pltpu
from jax.experimental.pallas import tpu_sc as plsc

BATCH = 16384
HIDDEN = 32

_NC = 2
_NS = 16
_NW = _NC * _NS
_BPW = BATCH // _NW
_CHUNK = 128
_NCHUNK = _BPW // _CHUNK
_LANES = 16


def _sc_body(trip_hbm, uemb, iemb, ibias,
             x_out, l2_out,
             trip_v, idxu_v, idxi_v, idxj_v,
             u_v, i_v, j_v, ib_v, jb_v, x_v, l2_v, sem):
    wid = lax.axis_index("s") * _NC + lax.axis_index("c")
    base = wid * _BPW

    pltpu.sync_copy(trip_hbm.at[pl.ds(base, _BPW)], trip_v)

    lane = lax.iota(jnp.int32, _LANES)
    for k in range(_BPW // _LANES):
        rows = lane + (k * _LANES)
        dst = pl.ds(k * _LANES, _LANES)
        idxu_v[dst] = plsc.load_gather(trip_v, [rows, jnp.zeros((_LANES,), jnp.int32)])
        idxi_v[dst] = plsc.load_gather(trip_v, [rows, jnp.full((_LANES,), 1, jnp.int32)])
        idxj_v[dst] = plsc.load_gather(trip_v, [rows, jnp.full((_LANES,), 2, jnp.int32)])

    copies = []
    for k in range(_NCHUNK):
        s = pl.ds(k * _CHUNK, _CHUNK)
        copies.append(pltpu.async_copy(uemb.at[idxu_v.at[s]], u_v.at[s], sem))
        copies.append(pltpu.async_copy(iemb.at[idxi_v.at[s]], i_v.at[s], sem))
        copies.append(pltpu.async_copy(iemb.at[idxj_v.at[s]], j_v.at[s], sem))
        copies.append(pltpu.async_copy(ibias.at[idxi_v.at[s]], ib_v.at[s], sem))
        copies.append(pltpu.async_copy(ibias.at[idxj_v.at[s]], jb_v.at[s], sem))
    for c in copies:
        c.wait()

    def step(k, l2acc):
        off = k * _LANES
        rows = lane + off
        acc = ib_v[pl.ds(off, _LANES)] - jb_v[pl.ds(off, _LANES)]
        for h in range(HIDDEN):
            hv = jnp.full((_LANES,), h, jnp.int32)
            uh = plsc.load_gather(u_v, [rows, hv])
            ih = plsc.load_gather(i_v, [rows, hv])
            jh = plsc.load_gather(j_v, [rows, hv])
            acc = acc + uh * (ih - jh)
            l2acc = l2acc + (uh * uh + ih * ih + jh * jh)
        x_v[pl.ds(off, _LANES)] = acc
        return l2acc

    l2acc = lax.fori_loop(0, _BPW // _LANES, step, jnp.zeros((_LANES,), jnp.float32))
    l2_v[...] = l2acc

    pltpu.sync_copy(x_v, x_out.at[pl.ds(base, _BPW)])
    pltpu.sync_copy(l2_v, l2_out.at[wid])


def _sc_call(trip, user_embedding, item_embedding, item_bias):
    mesh = plsc.VectorSubcoreMesh(core_axis_name="c", subcore_axis_name="s")
    f = functools.partial(
        pl.kernel,
        mesh=mesh,
        compiler_params=pltpu.CompilerParams(use_tc_tiling_on_sc=False,
                                             needs_layout_passes=False),
        out_type=(
            jax.ShapeDtypeStruct((BATCH,), jnp.float32),
            jax.ShapeDtypeStruct((_NW, _LANES), jnp.float32),
        ),
        scratch_types=[
            pltpu.VMEM((_BPW, 3), jnp.int32),
            pltpu.VMEM((_BPW,), jnp.int32),
            pltpu.VMEM((_BPW,), jnp.int32),
            pltpu.VMEM((_BPW,), jnp.int32),
            pltpu.VMEM((_BPW, HIDDEN), jnp.float32),
            pltpu.VMEM((_BPW, HIDDEN), jnp.float32),
            pltpu.VMEM((_BPW, HIDDEN), jnp.float32),
            pltpu.VMEM((_BPW,), jnp.float32),
            pltpu.VMEM((_BPW,), jnp.float32),
            pltpu.VMEM((_BPW,), jnp.float32),
            pltpu.VMEM((_LANES,), jnp.float32),
            pltpu.SemaphoreType.DMA,
        ],
    )(_sc_body)
    return f(trip, user_embedding, item_embedding, item_bias)


def _loss_body(x_ref, l2_ref, out_ref):
    x = x_ref[...]
    ls = jnp.minimum(x, 0.0) - jnp.log(1.0 + jnp.exp(-jnp.abs(x)))
    l2 = jnp.sum(l2_ref[...])
    out_ref[0, 0] = 0.0001 * l2 - jnp.mean(ls)


def _tc_loss(x, l2p):
    return pl.pallas_call(
        _loss_body,
        out_shape=jax.ShapeDtypeStruct((1, 1), jnp.float32),
        out_specs=pl.BlockSpec(memory_space=pltpu.SMEM),
    )(x, l2p)


def kernel(input, user_embedding, item_embedding, item_bias):
    x, l2p = _sc_call(input.astype(jnp.int32),
                      user_embedding, item_embedding, item_bias)
    return _tc_loss(x, l2p).reshape(())
